# baseline (speedup 1.0000x reference)
import jax
import jax.numpy as jnp
from jax import lax
from jax.experimental import pallas as pl
from jax.experimental.pallas import tpu as pltpu
from jax.experimental.pallas import tpu_sc as plsc

N_SENT = 4096
S = 50
SP = 56
D = 128
NC, NS = 2, 16
NW = NC * NS
SENT_PER_W = N_SENT // NW
SENT_PER_CHUNK = 1
C = SENT_PER_CHUNK * S
NCHUNK = SENT_PER_W // SENT_PER_CHUNK


def _emb_body(idx_hbm, table_hbm, out_hbm, idx_v, buf0, buf1, g0, g1, o0, o1):
    wid = lax.axis_index("s") * NC + lax.axis_index("c")
    pltpu.sync_copy(idx_hbm.at[wid], idx_v)
    sent_base = wid * SENT_PER_W

    bufs = (buf0, buf1)
    gsems = (g0, g1)
    osems = (o0, o1)

    def gather(c, b):
        pltpu.async_copy(table_hbm.at[idx_v.at[c]], bufs[b], gsems[b])

    def wait_gather(c, b):
        pltpu.make_async_copy(table_hbm.at[idx_v.at[c]], bufs[b], gsems[b]).wait()

    def store(c, b):
        pltpu.async_copy(bufs[b], out_hbm.at[sent_base + c], osems[b])

    def wait_store(b):
        pltpu.make_async_copy(bufs[b], out_hbm.at[0], osems[b]).wait()

    gather(0, 0)

    def body(g, carry):
        c0 = g * 2
        wait_gather(c0, 0)
        store(c0, 0)

        @pl.when(g >= 1)
        def _():
            wait_store(1)

        gather(c0 + 1, 1)

        wait_gather(c0 + 1, 1)
        store(c0 + 1, 1)
        wait_store(0)

        @pl.when(g < NCHUNK // 2 - 1)
        def _():
            gather(c0 + 2, 0)

        return carry

    lax.fori_loop(0, NCHUNK // 2, body, 0)
    wait_store(1)


def _run(idx_grp, table):
    f = pl.kernel(
        _emb_body,
        out_type=jax.ShapeDtypeStruct((N_SENT, S, D), jnp.float32),
        mesh=plsc.VectorSubcoreMesh(core_axis_name="c", subcore_axis_name="s"),
        compiler_params=pltpu.CompilerParams(use_tc_tiling_on_sc=True),
        scratch_types=[
            pltpu.VMEM((NCHUNK, C), jnp.int32),
            pltpu.VMEM((C, D), jnp.float32),
            pltpu.VMEM((C, D), jnp.float32),
            pltpu.SemaphoreType.DMA,
            pltpu.SemaphoreType.DMA,
            pltpu.SemaphoreType.DMA,
            pltpu.SemaphoreType.DMA,
        ],
    )
    return f(idx_grp, table)


def kernel(idx, emb_weight):
    idx_grp = idx.astype(jnp.int32).reshape(NW, NCHUNK, C)
    return _run(idx_grp, emb_weight)


# speedup vs baseline: 2.0718x; 2.0718x over previous
import jax
import jax.numpy as jnp
from jax import lax
from jax.experimental import pallas as pl
from jax.experimental.pallas import tpu as pltpu
from jax.experimental.pallas import tpu_sc as plsc

N_SENT = 4096
S = 50
SP = 56
D = 128
NC, NS = 2, 16
NW = NC * NS
SENT_PER_W = N_SENT // NW
B_ROWS = N_SENT * S
B_PER_W = B_ROWS // NW
C = 128
NCHUNK = B_PER_W // C


def _emb_body(idx_hbm, table_hbm, out_hbm, idx_v, buf0, buf1, g0, g1, o0, o1):
    wid = lax.axis_index("s") * NC + lax.axis_index("c")
    pltpu.sync_copy(idx_hbm.at[wid], idx_v)
    row_base = wid * B_PER_W

    bufs = (buf0, buf1)
    gsems = (g0, g1)
    osems = (o0, o1)

    def gather(c, b):
        pltpu.async_copy(table_hbm.at[idx_v.at[c]], bufs[b], gsems[b])

    def wait_gather(c, b):
        pltpu.make_async_copy(table_hbm.at[idx_v.at[c]], bufs[b], gsems[b]).wait()

    def store(c, b):
        pltpu.async_copy(
            bufs[b], out_hbm.at[pl.ds(row_base + c * C, C)], osems[b]
        )

    def wait_store(b):
        pltpu.make_async_copy(
            bufs[b], out_hbm.at[pl.ds(0, C)], osems[b]
        ).wait()

    gather(0, 0)

    def body(g, carry):
        c0 = g * 2
        wait_gather(c0, 0)
        store(c0, 0)

        @pl.when(g >= 1)
        def _():
            wait_store(1)

        gather(c0 + 1, 1)

        wait_gather(c0 + 1, 1)
        store(c0 + 1, 1)
        wait_store(0)

        @pl.when(g < NCHUNK // 2 - 1)
        def _():
            gather(c0 + 2, 0)

        return carry

    lax.fori_loop(0, NCHUNK // 2, body, 0)
    wait_store(1)


def _run(idx_grp, table):
    f = pl.kernel(
        _emb_body,
        out_type=jax.ShapeDtypeStruct((B_ROWS, D), jnp.float32),
        mesh=plsc.VectorSubcoreMesh(core_axis_name="c", subcore_axis_name="s"),
        scratch_types=[
            pltpu.VMEM((NCHUNK, C), jnp.int32),
            pltpu.VMEM((C, D), jnp.float32),
            pltpu.VMEM((C, D), jnp.float32),
            pltpu.SemaphoreType.DMA,
            pltpu.SemaphoreType.DMA,
            pltpu.SemaphoreType.DMA,
            pltpu.SemaphoreType.DMA,
        ],
    )
    return f(idx_grp, table)


def kernel(idx, emb_weight):
    idx_grp = idx.astype(jnp.int32).T.reshape(NW, NCHUNK, C)
    out = _run(idx_grp, emb_weight)
    return out.reshape(S, N_SENT, D).transpose(1, 0, 2)


# speedup vs baseline: 2.4846x; 1.1992x over previous
import jax
import jax.numpy as jnp
from jax import lax
from jax.experimental import pallas as pl
from jax.experimental.pallas import tpu as pltpu
from jax.experimental.pallas import tpu_sc as plsc

N_SENT = 4096
S = 50
SP = 56
D = 128
NC, NS = 2, 16
NW = NC * NS
SENT_PER_W = N_SENT // NW
B_ROWS = N_SENT * S
B_PER_W = B_ROWS // NW
C = 128
NCHUNK = B_PER_W // C


NBUF = 5


def _emb_body(idx_hbm, table_hbm, out_hbm, idx_v,
              buf0, buf1, buf2, buf3, buf4,
              g0, g1, g2, g3, g4, o0, o1, o2, o3, o4):
    wid = lax.axis_index("s") * NC + lax.axis_index("c")
    pltpu.sync_copy(idx_hbm.at[wid], idx_v)
    row_base = wid * B_PER_W

    bufs = (buf0, buf1, buf2, buf3, buf4)
    gsems = (g0, g1, g2, g3, g4)
    osems = (o0, o1, o2, o3, o4)

    def gather(c, b):
        pltpu.async_copy(table_hbm.at[idx_v.at[c]], bufs[b], gsems[b])

    def wait_gather(c, b):
        pltpu.make_async_copy(table_hbm.at[idx_v.at[c]], bufs[b], gsems[b]).wait()

    def store(c, b):
        pltpu.async_copy(
            bufs[b], out_hbm.at[pl.ds(row_base + c * C, C)], osems[b]
        )

    def wait_store(b):
        pltpu.make_async_copy(
            bufs[b], out_hbm.at[pl.ds(0, C)], osems[b]
        ).wait()

    for b in range(3):
        gather(b, b)

    NG = NCHUNK // NBUF

    def body(g, carry):
        for b in range(NBUF):
            c = g * NBUF + b
            wait_gather(c, b)
            store(c, b)
            bw = (b - 2) % NBUF
            if b >= 2:
                wait_store(bw)
            else:
                @pl.when(g >= 1)
                def _():
                    wait_store(bw)
            if b < 2:
                gather(c + 3, (b + 3) % NBUF)
            else:
                @pl.when(g < NG - 1)
                def _():
                    gather(c + 3, (b + 3) % NBUF)
        return carry

    lax.fori_loop(0, NG, body, 0)
    wait_store(3)
    wait_store(4)


def _run(idx_grp, table):
    f = pl.kernel(
        _emb_body,
        out_type=jax.ShapeDtypeStruct((B_ROWS, D), jnp.float32),
        mesh=plsc.VectorSubcoreMesh(core_axis_name="c", subcore_axis_name="s"),
        scratch_types=(
            [pltpu.VMEM((NCHUNK, C), jnp.int32)]
            + [pltpu.VMEM((C, D), jnp.float32)] * NBUF
            + [pltpu.SemaphoreType.DMA] * (2 * NBUF)
        ),
    )
    return f(idx_grp, table)


def kernel(idx, emb_weight):
    idx_grp = idx.astype(jnp.int32).T.reshape(NW, NCHUNK, C)
    out = _run(idx_grp, emb_weight)
    return out.reshape(S, N_SENT, D).transpose(1, 0, 2)


# speedup vs baseline: 2.4985x; 1.0056x over previous
import jax
import jax.numpy as jnp
from jax import lax
from jax.experimental import pallas as pl
from jax.experimental.pallas import tpu as pltpu
from jax.experimental.pallas import tpu_sc as plsc

N_SENT = 4096
S = 50
D = 128
NC, NS = 2, 16
NW = NC * NS
B_ROWS = N_SENT * S
B_PER_W = B_ROWS // NW
C = 80
NCHUNK = B_PER_W // C
NBUF = 10
DIST = 6


def _emb_body(idx_hbm, table_hbm, out_hbm, idx_v, *rest):
    wid = lax.axis_index("s") * NC + lax.axis_index("c")
    pltpu.sync_copy(idx_hbm.at[wid], idx_v)
    row_base = wid * B_PER_W

    bufs = rest[:NBUF]
    gsems = rest[NBUF:2 * NBUF]
    osems = rest[2 * NBUF:3 * NBUF]

    def gather(c, b):
        pltpu.async_copy(table_hbm.at[idx_v.at[c]], bufs[b], gsems[b])

    def wait_gather(c, b):
        pltpu.make_async_copy(table_hbm.at[idx_v.at[c]], bufs[b], gsems[b]).wait()

    def store(c, b):
        pltpu.async_copy(
            bufs[b], out_hbm.at[pl.ds(row_base + c * C, C)], osems[b]
        )

    def wait_store(b):
        pltpu.make_async_copy(
            bufs[b], out_hbm.at[pl.ds(0, C)], osems[b]
        ).wait()

    for b in range(DIST):
        gather(b, b)

    NG = NCHUNK // NBUF

    def body(g, carry):
        for b in range(NBUF):
            c = g * NBUF + b
            wait_gather(c, b)
            store(c, b)
            bw = (b + DIST) % NBUF
            if b >= NBUF - DIST:
                wait_store(bw)
            else:
                @pl.when(g >= 1)
                def _():
                    wait_store(bw)
            if b < NBUF - DIST:
                gather(c + DIST, bw)
            else:
                @pl.when(g < NG - 1)
                def _():
                    gather(c + DIST, bw)
        return carry

    lax.fori_loop(0, NG, body, 0)
    for b in range(DIST, NBUF):
        wait_store(b)


def _run(idx_grp, table):
    f = pl.kernel(
        _emb_body,
        out_type=jax.ShapeDtypeStruct((B_ROWS, D), jnp.float32),
        mesh=plsc.VectorSubcoreMesh(core_axis_name="c", subcore_axis_name="s"),
        scratch_types=(
            [pltpu.VMEM((NCHUNK, C), jnp.int32)]
            + [pltpu.VMEM((C, D), jnp.float32)] * NBUF
            + [pltpu.SemaphoreType.DMA] * (2 * NBUF)
        ),
    )
    return f(idx_grp, table)


def kernel(idx, emb_weight):
    idx_grp = idx.astype(jnp.int32).T.reshape(NW, NCHUNK, C)
    out = _run(idx_grp, emb_weight)
    return out.reshape(S, N_SENT, D).transpose(1, 0, 2)


# speedup vs baseline: 2.5108x; 1.0050x over previous
import jax
import jax.numpy as jnp
from jax import lax
from jax.experimental import pallas as pl
from jax.experimental.pallas import tpu as pltpu
from jax.experimental.pallas import tpu_sc as plsc

N_SENT = 4096
S = 50
D = 128
NC, NS = 2, 16
NW = NC * NS
B_ROWS = N_SENT * S
B_PER_W = B_ROWS // NW
C = 80
NCHUNK = B_PER_W // C
NBUF = 10
DIST = 7


def _emb_body(idx_hbm, table_hbm, out_hbm, idx_v, *rest):
    wid = lax.axis_index("s") * NC + lax.axis_index("c")
    pltpu.sync_copy(idx_hbm.at[wid], idx_v)
    row_base = wid * B_PER_W

    bufs = rest[:NBUF]
    gsems = rest[NBUF:2 * NBUF]
    osems = rest[2 * NBUF:3 * NBUF]

    def gather(c, b):
        pltpu.async_copy(table_hbm.at[idx_v.at[c]], bufs[b], gsems[b])

    def wait_gather(c, b):
        pltpu.make_async_copy(table_hbm.at[idx_v.at[c]], bufs[b], gsems[b]).wait()

    def store(c, b):
        pltpu.async_copy(
            bufs[b], out_hbm.at[pl.ds(row_base + c * C, C)], osems[b]
        )

    def wait_store(b):
        pltpu.make_async_copy(
            bufs[b], out_hbm.at[pl.ds(0, C)], osems[b]
        ).wait()

    for b in range(DIST):
        gather(b, b)

    NG = NCHUNK // NBUF

    def body(g, carry):
        for b in range(NBUF):
            c = g * NBUF + b
            wait_gather(c, b)
            store(c, b)
            bw = (b + DIST) % NBUF
            if b >= NBUF - DIST:
                wait_store(bw)
            else:
                @pl.when(g >= 1)
                def _():
                    wait_store(bw)
            if b < NBUF - DIST:
                gather(c + DIST, bw)
            else:
                @pl.when(g < NG - 1)
                def _():
                    gather(c + DIST, bw)
        return carry

    lax.fori_loop(0, NG, body, 0)
    for b in range(DIST, NBUF):
        wait_store(b)


def _run(idx_grp, table):
    f = pl.kernel(
        _emb_body,
        out_type=jax.ShapeDtypeStruct((B_ROWS, D), jnp.float32),
        mesh=plsc.VectorSubcoreMesh(core_axis_name="c", subcore_axis_name="s"),
        scratch_types=(
            [pltpu.VMEM((NCHUNK, C), jnp.int32)]
            + [pltpu.VMEM((C, D), jnp.float32)] * NBUF
            + [pltpu.SemaphoreType.DMA] * (2 * NBUF)
        ),
    )
    return f(idx_grp, table)


def kernel(idx, emb_weight):
    idx_grp = idx.astype(jnp.int32).T.reshape(NW, NCHUNK, C)
    out = _run(idx_grp, emb_weight)
    return out.reshape(S, N_SENT, D).transpose(1, 0, 2)
